# Initial kernel scaffold; baseline (speedup 1.0000x reference)
import jax
import jax.numpy as jnp
from jax.experimental import pallas as pl


def kernel(x_src, x_dst, edge_attr, edge_index, batch_size, W_emb, b_emb, lns_g, lns_b, lnd_g, lnd_b, Wq, bq, Wk, bk, Wv, bv, We, be, Wo, bo, ln2_g, ln2_b, W1, b1, W2, b2):
    raise NotImplementedError("write your pallas kernel here")


# speedup vs baseline: 7.9881x; 7.9881x over previous
import functools

import jax
import jax.numpy as jnp
from jax import lax
from jax.experimental import pallas as pl
from jax.experimental.pallas import tpu as pltpu
from jax.experimental.pallas import tpu_sc as plsc

NC = 2
NS = 16
LANES = 16
ACC_W = 144


def _ln(x, g, b):
  m = jnp.mean(x, axis=-1, keepdims=True)
  v = jnp.mean((x - m) ** 2, axis=-1, keepdims=True)
  return (x - m) / jnp.sqrt(v + 1e-5) * g + b


def _prep_body(xsrc, xdst, wemb, bemb, lnsg, lnsb, lndg, lndb,
               wq, bq, wk, bk, wv, bv, scale_ref,
               q_out, k_out, v_out, xde_out):
  xde = jnp.dot(xdst[...], wemb[...], preferred_element_type=jnp.float32) + bemb[...]
  xde_out[...] = xde
  xs = _ln(xsrc[...], lnsg[...], lnsb[...])
  xd = _ln(xde, lndg[...], lndb[...])
  s = scale_ref[0, 0]
  q_out[...] = (jnp.dot(xd, wq[...], preferred_element_type=jnp.float32) + bq[...]) * s
  k_out[...] = jnp.dot(xs, wk[...], preferred_element_type=jnp.float32) + bk[...]
  v_out[...] = jnp.dot(xs, wv[...], preferred_element_type=jnp.float32) + bv[...]


def _prep(x_src, x_dst, W_emb, b_emb, lns_g, lns_b, lnd_g, lnd_b,
          Wq, bq, Wk, bk, Wv, bv, scale):
  n, hid = x_src.shape
  blk = 1000
  grid = (n // blk,)
  row = lambda i: (i, 0)
  fix = lambda i: (0, 0)
  vec = lambda a: a.reshape(1, -1)
  full = lambda shp: pl.BlockSpec(shp, fix)
  out_shape = [jax.ShapeDtypeStruct((n, hid), jnp.float32)] * 4
  return pl.pallas_call(
      _prep_body,
      grid=grid,
      in_specs=[pl.BlockSpec((blk, hid), row), pl.BlockSpec((blk, hid), row),
                full((hid, hid)), full((1, hid)), full((1, hid)), full((1, hid)),
                full((1, hid)), full((1, hid)), full((hid, hid)), full((1, hid)),
                full((hid, hid)), full((1, hid)), full((hid, hid)), full((1, hid)),
                full((1, 1))],
      out_specs=[pl.BlockSpec((blk, hid), row)] * 4,
      out_shape=out_shape,
  )(x_src, x_dst, W_emb, vec(b_emb), vec(lns_g), vec(lns_b), vec(lnd_g),
    vec(lnd_b), Wq, vec(bq), Wk, vec(bk), Wv, vec(bv), scale.reshape(1, 1))


def _eproj_body(ea, bd, be, e_out):
  e_out[...] = jnp.dot(ea[...], bd[...], preferred_element_type=jnp.float32) + be[...]


def _eproj(ea_packed, BD, be_tiled):
  rows, kdim = ea_packed.shape
  odim = BD.shape[1]
  blk = 1000
  return pl.pallas_call(
      _eproj_body,
      grid=(rows // blk,),
      in_specs=[pl.BlockSpec((blk, kdim), lambda i: (i, 0)),
                pl.BlockSpec((kdim, odim), lambda i: (0, 0)),
                pl.BlockSpec((1, odim), lambda i: (0, 0))],
      out_specs=pl.BlockSpec((blk, odim), lambda i: (i, 0)),
      out_shape=jax.ShapeDtypeStruct((rows, odim), jnp.float32),
  )(ea_packed, BD, be_tiled)


def _tree_sum(terms):
  while len(terms) > 1:
    nxt = [terms[i] + terms[i + 1] for i in range(0, len(terms) - 1, 2)]
    if len(terms) % 2:
      nxt.append(terms[-1])
    terms = nxt
  return terms[0]


def _make_edge_kernel(n_dst, n_edges, hid, heads, chunk):
  dh = hid // heads
  nchunks_total = n_edges // chunk
  ntiles = NC * NS
  nch_base = nchunks_total // ntiles
  nch_rem = nchunks_total % ntiles
  nrows = ((n_dst + NS - 1) // NS + 7) // 8 * 8
  acc_n = nrows * NS
  ndrows = (((n_dst + 7) // 8 + NS - 1) // NS + 7) // 8 * 8
  acc_nd = ndrows * NS
  groups = chunk // LANES

  mesh = plsc.VectorSubcoreMesh(
      core_axis_name="c", subcore_axis_name="s", num_cores=NC, num_subcores=NS)

  def body(sidx_hbm, didx_hbm, q_hbm, k_hbm, v_hbm, e_hbm,
           pnum_hbm, pden_hbm,
           acc_num, acc_den, sidx_v, didx_v, didx8_v, qbuf, kbuf, vbuf, ebuf,
           aden, sem_k, sem_v, sem_q, sem_e):
    c = lax.axis_index("c")
    s = lax.axis_index("s")
    tile = c * NS + s
    nch = nch_base + jnp.where(tile < nch_rem, 1, 0)
    r0 = s * nrows
    d0 = s * ndrows
    z16 = jnp.zeros((LANES,), jnp.float32)

    def _zq(i, carry):
      for j in range(hid // LANES):
        qbuf[i, pl.ds(j * LANES, LANES)] = z16
        aden[i, pl.ds(j * LANES, LANES)] = z16
      return carry
    lax.fori_loop(0, chunk, _zq, 0)

    def _stripes(dst_at, base, total, src):
      nfull = total // chunk
      def _zs(i, carry):
        pltpu.sync_copy(src, dst_at(base + i * chunk, chunk))
        return carry
      lax.fori_loop(0, nfull, _zs, 0)
      rem = total - nfull * chunk
      off = base + nfull * chunk
      while rem >= 8:
        step = 16 if rem >= 16 else 8
        pltpu.sync_copy(src.at[pl.ds(0, step)] if hasattr(src, "at") else src,
                        dst_at(off, step))
        off += step
        rem -= step

    _stripes(lambda r, n: acc_num.at[pl.ds(r, n)], r0, nrows, qbuf)
    _stripes(lambda r, n: acc_den.at[pl.ds(r, n)], d0, ndrows, aden)

    plsc.subcore_barrier()

    def _chunk(ci, carry):
      eb = (tile + ci * ntiles) * chunk
      pltpu.sync_copy(sidx_hbm.at[pl.ds(eb, chunk)], sidx_v)
      pltpu.sync_copy(didx_hbm.at[pl.ds(eb, chunk)], didx_v)
      cpk = pltpu.async_copy(k_hbm.at[sidx_v], kbuf, sem_k)
      cpv = pltpu.async_copy(v_hbm.at[sidx_v], vbuf, sem_v)
      cpq = pltpu.async_copy(q_hbm.at[didx_v], qbuf, sem_q)
      cpe = pltpu.async_copy(e_hbm.at[pl.ds(eb, chunk)], ebuf, sem_e)

      def _za(i, carry2):
        for j in range(hid // LANES):
          aden[i, pl.ds(j * LANES, LANES)] = z16
        return carry2
      lax.fori_loop(0, chunk, _za, 0)

      cpk.wait(); cpv.wait(); cpq.wait(); cpe.wait()

      def _g16(g, carry2):
        iv = didx_v[pl.ds(g * LANES, LANES)]
        didx8_v[pl.ds(g * LANES, LANES)] = lax.shift_right_logical(iv, 3)
        return carry2
      lax.fori_loop(0, groups, _g16, 0)

      def _group(g, inner):
        rows = g * LANES + lax.iota(jnp.int32, LANES)
        iv = didx_v[pl.ds(g * LANES, LANES)]
        lanebase = (iv & 7) * 16
        for h in range(heads):
          terms = []
          for j in range(dh):
            d = h * dh + j
            cold = jnp.full((LANES,), d, jnp.int32)
            qd = plsc.load_gather(qbuf, [rows, cold])
            kd = plsc.load_gather(kbuf, [rows, cold])
            ed = plsc.load_gather(ebuf, [rows, cold])
            terms.append(qd * (kd + ed))
          ex = jnp.exp(_tree_sum(terms))
          plsc.store_scatter(aden, [rows, lanebase + h], ex)
          for j in range(dh):
            d = h * dh + j
            cold = jnp.full((LANES,), d, jnp.int32)
            vd = plsc.load_gather(vbuf, [rows, cold])
            ed = plsc.load_gather(ebuf, [rows, cold])
            plsc.store_scatter(ebuf, [rows, cold], (vd + ed) * ex)
        return inner
      lax.fori_loop(0, groups, _group, 0)

      pltpu.sync_copy(ebuf, acc_num.at[didx_v], add=True)
      pltpu.sync_copy(aden, acc_den.at[didx8_v], add=True)
      return carry
    lax.fori_loop(0, nch, _chunk, 0)

    plsc.subcore_barrier()

    def _wout(acc, out_at, base, total, buf):
      nfull = total // chunk
      def _ws(i, carry):
        r = base + i * chunk
        pltpu.sync_copy(acc.at[pl.ds(r, chunk)], buf)
        pltpu.sync_copy(buf, out_at(r, chunk))
        return carry
      lax.fori_loop(0, nfull, _ws, 0)
      rem = total - nfull * chunk
      off = base + nfull * chunk
      while rem >= 8:
        step = 16 if rem >= 16 else 8
        pltpu.sync_copy(acc.at[pl.ds(off, step)], buf.at[pl.ds(0, step)])
        pltpu.sync_copy(buf.at[pl.ds(0, step)], out_at(off, step))
        off += step
        rem -= step

    _wout(acc_num, lambda r, n: pnum_hbm.at[c, pl.ds(r, n)], r0, nrows, qbuf)
    _wout(acc_den, lambda r, n: pden_hbm.at[c, pl.ds(r, n)], d0, ndrows, aden)

  f = pl.kernel(
      body,
      out_type=[jax.ShapeDtypeStruct((NC, acc_n, hid), jnp.float32),
                jax.ShapeDtypeStruct((NC, acc_nd, hid), jnp.float32)],
      mesh=mesh,
      compiler_params=pltpu.CompilerParams(needs_layout_passes=False),
      scratch_types=[
          pltpu.VMEM_SHARED((acc_n, hid), jnp.float32),
          pltpu.VMEM_SHARED((acc_nd, hid), jnp.float32),
          pltpu.VMEM((chunk,), jnp.int32),
          pltpu.VMEM((chunk,), jnp.int32),
          pltpu.VMEM((chunk,), jnp.int32),
          pltpu.VMEM((chunk, hid), jnp.float32),
          pltpu.VMEM((chunk, hid), jnp.float32),
          pltpu.VMEM((chunk, hid), jnp.float32),
          pltpu.VMEM((chunk, hid), jnp.float32),
          pltpu.VMEM((chunk, hid), jnp.float32),
          pltpu.SemaphoreType.DMA,
          pltpu.SemaphoreType.DMA,
          pltpu.SemaphoreType.DMA,
          pltpu.SemaphoreType.DMA,
      ],
  )
  return f


def _post_body(pnum, pden, xde, r16, wo, bo, ln2g, ln2b, w1, b1, w2, b2, out):
  w = pnum[0] + pnum[1]
  den16 = pden[0] + pden[1]
  denw = jnp.dot(den16, r16[...], preferred_element_type=jnp.float32)
  agg = w / (denw + 1e-16)
  x1 = xde[...] + jnp.dot(agg, wo[...], preferred_element_type=jnp.float32) + bo[...]
  h = _ln(x1, ln2g[...], ln2b[...])
  mlp = jnp.dot(jax.nn.gelu(jnp.dot(h, w1[...], preferred_element_type=jnp.float32) + b1[...]),
                w2[...], preferred_element_type=jnp.float32) + b2[...]
  out[...] = x1 + mlp


def _post(pnum, pden, xde, R16, Wo, bo, ln2_g, ln2_b, W1, b1, W2, b2):
  n, hid = xde.shape
  mlp_hid = W1.shape[1]
  blk = 1000
  fix = lambda i: (0, 0)
  vec = lambda a: a.reshape(1, -1)
  return pl.pallas_call(
      _post_body,
      grid=(n // blk,),
      in_specs=[pl.BlockSpec((NC, blk, hid), lambda i: (0, i, 0)),
                pl.BlockSpec((NC, blk, LANES), lambda i: (0, i, 0)),
                pl.BlockSpec((blk, hid), lambda i: (i, 0)),
                pl.BlockSpec((LANES, hid), fix),
                pl.BlockSpec((hid, hid), fix), pl.BlockSpec((1, hid), fix),
                pl.BlockSpec((1, hid), fix), pl.BlockSpec((1, hid), fix),
                pl.BlockSpec((hid, mlp_hid), fix), pl.BlockSpec((1, mlp_hid), fix),
                pl.BlockSpec((mlp_hid, hid), fix), pl.BlockSpec((1, hid), fix)],
      out_specs=pl.BlockSpec((blk, hid), lambda i: (i, 0)),
      out_shape=jax.ShapeDtypeStruct((n, hid), jnp.float32),
  )(pnum, pden, xde, R16, Wo, vec(bo), vec(ln2_g), vec(ln2_b), W1, vec(b1), W2, vec(b2))


def kernel(x_src, x_dst, edge_attr, edge_index, batch_size,
           W_emb, b_emb, lns_g, lns_b, lnd_g, lnd_b,
           Wq, bq, Wk, bk, Wv, bv, We, be, Wo, bo,
           ln2_g, ln2_b, W1, b1, W2, b2):
  n_src, hid = x_src.shape
  n_dst = x_dst.shape[0]
  n_edges, edge_dim = edge_attr.shape
  heads = 8
  dh = hid // heads
  pack = hid // edge_dim

  scale = jnp.float32(1.0) / jnp.sqrt(jnp.float32(dh))
  q, k, v, xde = _prep(x_src, x_dst, W_emb, b_emb, lns_g, lns_b, lnd_g, lnd_b,
                       Wq, bq, Wk, bk, Wv, bv, scale)

  BD = jnp.einsum("ij,ao->iajo", jnp.eye(pack, dtype=jnp.float32),
                  We).reshape(pack * edge_dim, pack * hid)
  be_tiled = jnp.tile(be, pack).reshape(1, pack * hid)
  e = _eproj(edge_attr.reshape(n_edges // pack, pack * edge_dim),
             BD, be_tiled).reshape(n_edges, hid)

  ei = edge_index.astype(jnp.int32)
  src_idx = ei[0]
  dst_idx = ei[1]
  edge_f = _make_edge_kernel(n_dst, n_edges, hid, heads, chunk=32)
  pnum, pden = edge_f(src_idx, dst_idx, q, k, v, e)
  den16 = pden.reshape(pden.shape[0], pden.shape[1] * (hid // 16), 16)

  cols = jnp.arange(hid) // dh
  R16 = (jnp.arange(16)[:, None] == cols[None, :]).astype(jnp.float32)

  return _post(pnum, den16, xde, R16, Wo, bo, ln2_g, ln2_b, W1, b1, W2, b2)


# speedup vs baseline: 8.1224x; 1.0168x over previous
import functools

import jax
import jax.numpy as jnp
from jax import lax
from jax.experimental import pallas as pl
from jax.experimental.pallas import tpu as pltpu
from jax.experimental.pallas import tpu_sc as plsc

NC = 2
NS = 16
LANES = 16
ACC_W = 144


def _ln(x, g, b):
  m = jnp.mean(x, axis=-1, keepdims=True)
  v = jnp.mean((x - m) ** 2, axis=-1, keepdims=True)
  return (x - m) / jnp.sqrt(v + 1e-5) * g + b


def _prep_body(xsrc, xdst, wemb, bemb, lnsg, lnsb, lndg, lndb,
               wq, bq, wk, bk, wv, bv, scale_ref,
               q_out, kv_out, xde_out):
  xde = jnp.dot(xdst[...], wemb[...], preferred_element_type=jnp.float32) + bemb[...]
  xde_out[...] = xde
  xs = _ln(xsrc[...], lnsg[...], lnsb[...])
  xd = _ln(xde, lndg[...], lndb[...])
  sc = scale_ref[0, 0]
  hid = xsrc.shape[1]
  q_out[...] = (jnp.dot(xd, wq[...], preferred_element_type=jnp.float32) + bq[...]) * sc
  kv_out[:, 0:hid] = jnp.dot(xs, wk[...], preferred_element_type=jnp.float32) + bk[...]
  kv_out[:, hid:2 * hid] = jnp.dot(xs, wv[...], preferred_element_type=jnp.float32) + bv[...]


def _prep(x_src, x_dst, W_emb, b_emb, lns_g, lns_b, lnd_g, lnd_b,
          Wq, bq, Wk, bk, Wv, bv, scale):
  n, hid = x_src.shape
  blk = 1000
  grid = (n // blk,)
  row = lambda i: (i, 0)
  fix = lambda i: (0, 0)
  vec = lambda a: a.reshape(1, -1)
  full = lambda shp: pl.BlockSpec(shp, fix)
  out_shape = [jax.ShapeDtypeStruct((n, hid), jnp.float32),
               jax.ShapeDtypeStruct((n, 2 * hid), jnp.float32),
               jax.ShapeDtypeStruct((n, hid), jnp.float32)]
  return pl.pallas_call(
      _prep_body,
      grid=grid,
      in_specs=[pl.BlockSpec((blk, hid), row), pl.BlockSpec((blk, hid), row),
                full((hid, hid)), full((1, hid)), full((1, hid)), full((1, hid)),
                full((1, hid)), full((1, hid)), full((hid, hid)), full((1, hid)),
                full((hid, hid)), full((1, hid)), full((hid, hid)), full((1, hid)),
                full((1, 1))],
      out_specs=[pl.BlockSpec((blk, hid), row), pl.BlockSpec((blk, 2 * hid), row),
                 pl.BlockSpec((blk, hid), row)],
      out_shape=out_shape,
  )(x_src, x_dst, W_emb, vec(b_emb), vec(lns_g), vec(lns_b), vec(lnd_g),
    vec(lnd_b), Wq, vec(bq), Wk, vec(bk), Wv, vec(bv), scale.reshape(1, 1))


def _eproj_body(ea, bd, be, e_out):
  e_out[...] = jnp.dot(ea[...], bd[...], preferred_element_type=jnp.float32) + be[...]


def _eproj(ea_packed, BD, be_tiled):
  rows, kdim = ea_packed.shape
  odim = BD.shape[1]
  blk = next(b for b in (1256, 1024, 1000, 628, 512, 256, 128, 64, 8)
             if rows % b == 0 and b % 8 == 0)
  return pl.pallas_call(
      _eproj_body,
      grid=(rows // blk,),
      in_specs=[pl.BlockSpec((blk, kdim), lambda i: (i, 0)),
                pl.BlockSpec((kdim, odim), lambda i: (0, 0)),
                pl.BlockSpec((1, odim), lambda i: (0, 0))],
      out_specs=pl.BlockSpec((blk, odim), lambda i: (i, 0)),
      out_shape=jax.ShapeDtypeStruct((rows, odim), jnp.float32),
  )(ea_packed, BD, be_tiled)


def _tree_sum(terms):
  while len(terms) > 1:
    nxt = [terms[i] + terms[i + 1] for i in range(0, len(terms) - 1, 2)]
    if len(terms) % 2:
      nxt.append(terms[-1])
    terms = nxt
  return terms[0]


def _make_edge_kernel(n_dst, n_edges_pad, hid, heads, chunk):
  dh = hid // heads
  ntiles = NC * NS
  nch = n_edges_pad // (ntiles * chunk)
  nrows = ((n_dst + 1 + NS - 1) // NS + 7) // 8 * 8
  acc_n = nrows * NS
  ndrows = (((n_dst + 1 + 15) // 16 + NS - 1) // NS + 7) // 8 * 8
  acc_nd = ndrows * NS
  groups = chunk // LANES

  mesh = plsc.VectorSubcoreMesh(
      core_axis_name="c", subcore_axis_name="s", num_cores=NC, num_subcores=NS)

  def body(sidx_hbm, didx_hbm, q_hbm, kv_hbm, e_hbm,
           pnum_hbm, pden_hbm,
           acc_num, acc_den,
           sidx0, sidx1, didx0, didx1, sdidx0, sdidx1, sd16_0, sd16_1,
           kv0, kv1, q0, q1, e0, e1, aden,
           sem_kv0, sem_kv1, sem_q0, sem_q1, sem_e0, sem_e1,
           sem_n0, sem_n1, sem_d):
    c = lax.axis_index("c")
    s = lax.axis_index("s")
    tile = c * NS + s
    r0 = s * nrows
    d0 = s * ndrows
    z16 = jnp.zeros((LANES,), jnp.float32)
    sidx = (sidx0, sidx1)
    didx = (didx0, didx1)
    sdidx = (sdidx0, sdidx1)
    sd16 = (sd16_0, sd16_1)
    kvb = (kv0, kv1)
    qb = (q0, q1)
    eb_ = (e0, e1)
    sem_kv = (sem_kv0, sem_kv1)
    sem_q = (sem_q0, sem_q1)
    sem_e = (sem_e0, sem_e1)
    sem_n = (sem_n0, sem_n1)

    def _zb(i, carry):
      for j in range(hid // LANES):
        q0[i, pl.ds(j * LANES, LANES)] = z16
        aden[i, pl.ds(j * LANES, LANES)] = z16
      return carry
    lax.fori_loop(0, chunk, _zb, 0)

    def _stripes(dst_at, base, total, src):
      nfull = total // chunk
      def _zs(i, carry):
        pltpu.sync_copy(src, dst_at(base + i * chunk, chunk))
        return carry
      lax.fori_loop(0, nfull, _zs, 0)
      rem = total - nfull * chunk
      off = base + nfull * chunk
      while rem >= 8:
        step = 16 if rem >= 16 else 8
        pltpu.sync_copy(src.at[pl.ds(0, step)], dst_at(off, step))
        off += step
        rem -= step

    _stripes(lambda r, n: acc_num.at[pl.ds(r, n)], r0, nrows, q0)
    _stripes(lambda r, n: acc_den.at[pl.ds(r, n)], d0, ndrows, aden)

    plsc.subcore_barrier()

    def _eb(i):
      return (tile + i * ntiles) * chunk

    def _idx_copy(i, b):
      pltpu.sync_copy(sidx_hbm.at[pl.ds(_eb(i), chunk)], sidx[b])
      pltpu.sync_copy(didx_hbm.at[pl.ds(_eb(i), chunk)], didx[b])

    def _issue_kvq(i, b):
      pltpu.async_copy(kv_hbm.at[sidx[b]], kvb[b], sem_kv[b])
      pltpu.async_copy(q_hbm.at[didx[b]], qb[b], sem_q[b])

    def _issue_e(i, b):
      pltpu.async_copy(e_hbm.at[pl.ds(_eb(i), chunk)], eb_[b], sem_e[b])

    _idx_copy(0, 0)
    _issue_kvq(0, 0)
    _issue_e(0, 0)
    _idx_copy(1, 1)
    _issue_kvq(1, 1)
    _issue_e(1, 1)

    def _outer(ci2, carry):
      for b in (0, 1):
        j = ci2 * 2 + b
        pltpu.make_async_copy(kv_hbm.at[sidx[b]], kvb[b], sem_kv[b]).wait()
        pltpu.make_async_copy(q_hbm.at[didx[b]], qb[b], sem_q[b]).wait()
        pltpu.make_async_copy(e_hbm.at[pl.ds(0, chunk)], eb_[b], sem_e[b]).wait()
        @pl.when(j >= 1)
        def _wd():
          pltpu.make_async_copy(aden, acc_den.at[sd16[1 - b]], sem_d).wait()
        def _za(i, carry2):
          for jj in range(hid // LANES):
            aden[i, pl.ds(jj * LANES, LANES)] = z16
          return carry2
        lax.fori_loop(0, chunk, _za, 0)
        def _sidx(g, carry2):
          iv = didx[b][pl.ds(g * LANES, LANES)]
          sdidx[b][pl.ds(g * LANES, LANES)] = iv
          sd16[b][pl.ds(g * LANES, LANES)] = lax.shift_right_logical(iv, 4)
          return carry2
        lax.fori_loop(0, groups, _sidx, 0)
        def _group(g, inner):
          rows = g * LANES + lax.iota(jnp.int32, LANES)
          iv = didx[b][pl.ds(g * LANES, LANES)]
          lanebase = (iv & 15) * 8
          for h in range(heads):
            terms = []
            for jj in range(dh):
              d = h * dh + jj
              cold = jnp.full((LANES,), d, jnp.int32)
              qd = plsc.load_gather(qb[b], [rows, cold])
              kd = plsc.load_gather(kvb[b], [rows, cold])
              ed = plsc.load_gather(eb_[b], [rows, cold])
              terms.append(qd * (kd + ed))
            ex = jnp.exp(_tree_sum(terms))
            plsc.store_scatter(aden, [rows, lanebase + h], ex)
            for jj in range(dh):
              d = h * dh + jj
              cold = jnp.full((LANES,), d, jnp.int32)
              vd = plsc.load_gather(kvb[b], [rows, cold + hid])
              ed = plsc.load_gather(eb_[b], [rows, cold])
              plsc.store_scatter(eb_[b], [rows, cold], (vd + ed) * ex)
          return inner
        lax.fori_loop(0, groups, _group, 0)
        pltpu.async_copy(eb_[b], acc_num.at[sdidx[b]], sem_n[b], add=True)
        pltpu.async_copy(aden, acc_den.at[sd16[b]], sem_d, add=True)
        @pl.when(j >= 1)
        def _wn():
          pltpu.make_async_copy(eb_[1 - b], acc_num.at[sdidx[1 - b]],
                                sem_n[1 - b]).wait()
        @pl.when(j + 2 < nch)
        def _st():
          _idx_copy(j + 2, b)
          _issue_kvq(j + 2, b)
        @pl.when((j >= 1) & (j + 1 < nch))
        def _se():
          _issue_e(j + 1, 1 - b)
      return carry

    lax.fori_loop(0, nch // 2, _outer, 0)

    lastb = (nch - 1) & 1
    pltpu.make_async_copy(eb_[lastb], acc_num.at[sdidx[lastb]], sem_n[lastb]).wait()
    pltpu.make_async_copy(aden, acc_den.at[sd16[lastb]], sem_d).wait()

    plsc.subcore_barrier()

    def _wout(acc, out_at, base, total, buf):
      nfull = total // chunk
      def _ws(i, carry):
        r = base + i * chunk
        pltpu.sync_copy(acc.at[pl.ds(r, chunk)], buf)
        pltpu.sync_copy(buf, out_at(r, chunk))
        return carry
      lax.fori_loop(0, nfull, _ws, 0)
      rem = total - nfull * chunk
      off = base + nfull * chunk
      while rem >= 8:
        step = 16 if rem >= 16 else 8
        pltpu.sync_copy(acc.at[pl.ds(off, step)], buf.at[pl.ds(0, step)])
        pltpu.sync_copy(buf.at[pl.ds(0, step)], out_at(off, step))
        off += step
        rem -= step

    _wout(acc_num, lambda r, n: pnum_hbm.at[c, pl.ds(r, n)], r0, nrows, q0)
    _wout(acc_den, lambda r, n: pden_hbm.at[c, pl.ds(r, n)], d0, ndrows, aden)

  f = pl.kernel(
      body,
      out_type=[jax.ShapeDtypeStruct((NC, acc_n, hid), jnp.float32),
                jax.ShapeDtypeStruct((NC, acc_nd, hid), jnp.float32)],
      mesh=mesh,
      compiler_params=pltpu.CompilerParams(needs_layout_passes=False),
      scratch_types=[
          pltpu.VMEM_SHARED((acc_n, hid), jnp.float32),
          pltpu.VMEM_SHARED((acc_nd, hid), jnp.float32),
          pltpu.VMEM((chunk,), jnp.int32), pltpu.VMEM((chunk,), jnp.int32),
          pltpu.VMEM((chunk,), jnp.int32), pltpu.VMEM((chunk,), jnp.int32),
          pltpu.VMEM((chunk,), jnp.int32), pltpu.VMEM((chunk,), jnp.int32),
          pltpu.VMEM((chunk,), jnp.int32), pltpu.VMEM((chunk,), jnp.int32),
          pltpu.VMEM((chunk, 2 * hid), jnp.float32),
          pltpu.VMEM((chunk, 2 * hid), jnp.float32),
          pltpu.VMEM((chunk, hid), jnp.float32),
          pltpu.VMEM((chunk, hid), jnp.float32),
          pltpu.VMEM((chunk, hid), jnp.float32),
          pltpu.VMEM((chunk, hid), jnp.float32),
          pltpu.VMEM((chunk, hid), jnp.float32),
          pltpu.SemaphoreType.DMA, pltpu.SemaphoreType.DMA,
          pltpu.SemaphoreType.DMA, pltpu.SemaphoreType.DMA,
          pltpu.SemaphoreType.DMA, pltpu.SemaphoreType.DMA,
          pltpu.SemaphoreType.DMA, pltpu.SemaphoreType.DMA,
          pltpu.SemaphoreType.DMA,
      ],
  )
  return f


def _post_body(pnum, pden, xde, r8, wo, bo, ln2g, ln2b, w1, b1, w2, b2, out):
  w = pnum[0] + pnum[1]
  den8 = pden[0] + pden[1]
  denw = jnp.dot(den8, r8[...], preferred_element_type=jnp.float32)
  agg = w / (denw + 1e-16)
  x1 = xde[...] + jnp.dot(agg, wo[...], preferred_element_type=jnp.float32) + bo[...]
  h = _ln(x1, ln2g[...], ln2b[...])
  mlp = jnp.dot(jax.nn.gelu(jnp.dot(h, w1[...], preferred_element_type=jnp.float32) + b1[...]),
                w2[...], preferred_element_type=jnp.float32) + b2[...]
  out[...] = x1 + mlp


def _post(pnum, pden, xde, R8, Wo, bo, ln2_g, ln2_b, W1, b1, W2, b2):
  n, hid = xde.shape
  mlp_hid = W1.shape[1]
  blk = 1000
  fix = lambda i: (0, 0)
  vec = lambda a: a.reshape(1, -1)
  return pl.pallas_call(
      _post_body,
      grid=(n // blk,),
      in_specs=[pl.BlockSpec((NC, blk, hid), lambda i: (0, i, 0)),
                pl.BlockSpec((NC, blk, 8), lambda i: (0, i, 0)),
                pl.BlockSpec((blk, hid), lambda i: (i, 0)),
                pl.BlockSpec((8, hid), fix),
                pl.BlockSpec((hid, hid), fix), pl.BlockSpec((1, hid), fix),
                pl.BlockSpec((1, hid), fix), pl.BlockSpec((1, hid), fix),
                pl.BlockSpec((hid, mlp_hid), fix), pl.BlockSpec((1, mlp_hid), fix),
                pl.BlockSpec((mlp_hid, hid), fix), pl.BlockSpec((1, hid), fix)],
      out_specs=pl.BlockSpec((blk, hid), lambda i: (i, 0)),
      out_shape=jax.ShapeDtypeStruct((n, hid), jnp.float32),
  )(pnum, pden, xde, R8, Wo, vec(bo), vec(ln2_g), vec(ln2_b), W1, vec(b1), W2, vec(b2))


def kernel(x_src, x_dst, edge_attr, edge_index, batch_size,
           W_emb, b_emb, lns_g, lns_b, lnd_g, lnd_b,
           Wq, bq, Wk, bk, Wv, bv, We, be, Wo, bo,
           ln2_g, ln2_b, W1, b1, W2, b2):
  n_src, hid = x_src.shape
  n_dst = x_dst.shape[0]
  n_edges, edge_dim = edge_attr.shape
  heads = 8
  dh = hid // heads
  pack = hid // edge_dim
  chunk = 32
  ntiles = NC * NS

  nch = -(-n_edges // (ntiles * chunk))
  if nch % 2:
    nch += 1
  n_edges_pad = nch * ntiles * chunk

  scale = jnp.float32(1.0) / jnp.sqrt(jnp.float32(dh))
  q, kv, xde = _prep(x_src, x_dst, W_emb, b_emb, lns_g, lns_b, lnd_g, lnd_b,
                     Wq, bq, Wk, bk, Wv, bv, scale)
  q = jnp.pad(q, ((0, 8), (0, 0)))
  kv = jnp.pad(kv, ((0, 8), (0, 0)))

  BD = jnp.einsum("ij,ao->iajo", jnp.eye(pack, dtype=jnp.float32),
                  We).reshape(pack * edge_dim, pack * hid)
  be_tiled = jnp.tile(be, pack).reshape(1, pack * hid)
  ea_packed = edge_attr.reshape(n_edges // pack, pack * edge_dim)
  ea_packed = jnp.pad(ea_packed, ((0, (n_edges_pad - n_edges) // pack), (0, 0)))
  e = _eproj(ea_packed, BD, be_tiled).reshape(n_edges_pad, hid)

  ei = edge_index.astype(jnp.int32)
  src_idx = jnp.pad(ei[0], (0, n_edges_pad - n_edges))
  dst_idx = jnp.pad(ei[1], (0, n_edges_pad - n_edges),
                    constant_values=n_dst)

  edge_f = _make_edge_kernel(n_dst, n_edges_pad, hid, heads, chunk=chunk)
  pnum, pden = edge_f(src_idx, dst_idx, q, kv, e)
  den8 = pden.reshape(pden.shape[0], pden.shape[1] * (hid // 8), 8)

  cols = jnp.arange(hid) // dh
  R8 = (jnp.arange(8)[:, None] == cols[None, :]).astype(jnp.float32)

  return _post(pnum, den8, xde, R8, Wo, bo, ln2_g, ln2_b, W1, b1, W2, b2)


# speedup vs baseline: 15.0985x; 1.8589x over previous
import functools

import jax
import jax.numpy as jnp
from jax import lax
from jax.experimental import pallas as pl
from jax.experimental.pallas import tpu as pltpu
from jax.experimental.pallas import tpu_sc as plsc

NC = 2
NS = 16
LANES = 16
ACC_W = 144


def _ln(x, g, b):
  m = jnp.mean(x, axis=-1, keepdims=True)
  v = jnp.mean((x - m) ** 2, axis=-1, keepdims=True)
  return (x - m) / jnp.sqrt(v + 1e-5) * g + b


def _prep_body(xsrc, xdst, wemb, bemb, lnsg, lnsb, lndg, lndb,
               wq, bq, wk, bk, wv, bv, scale_ref,
               q_out, kv_out, xde_out):
  xde = jnp.dot(xdst[...], wemb[...], preferred_element_type=jnp.float32) + bemb[...]
  xde_out[...] = xde
  xs = _ln(xsrc[...], lnsg[...], lnsb[...])
  xd = _ln(xde, lndg[...], lndb[...])
  sc = scale_ref[0, 0]
  hid = xsrc.shape[1]
  q_out[...] = (jnp.dot(xd, wq[...], preferred_element_type=jnp.float32) + bq[...]) * sc
  kv_out[:, 0:hid] = jnp.dot(xs, wk[...], preferred_element_type=jnp.float32) + bk[...]
  kv_out[:, hid:2 * hid] = jnp.dot(xs, wv[...], preferred_element_type=jnp.float32) + bv[...]


def _prep(x_src, x_dst, W_emb, b_emb, lns_g, lns_b, lnd_g, lnd_b,
          Wq, bq, Wk, bk, Wv, bv, scale):
  n, hid = x_src.shape
  blk = 1000
  grid = (n // blk,)
  row = lambda i: (i, 0)
  fix = lambda i: (0, 0)
  vec = lambda a: a.reshape(1, -1)
  full = lambda shp: pl.BlockSpec(shp, fix)
  out_shape = [jax.ShapeDtypeStruct((n, hid), jnp.float32),
               jax.ShapeDtypeStruct((n, 2 * hid), jnp.float32),
               jax.ShapeDtypeStruct((n, hid), jnp.float32)]
  return pl.pallas_call(
      _prep_body,
      grid=grid,
      in_specs=[pl.BlockSpec((blk, hid), row), pl.BlockSpec((blk, hid), row),
                full((hid, hid)), full((1, hid)), full((1, hid)), full((1, hid)),
                full((1, hid)), full((1, hid)), full((hid, hid)), full((1, hid)),
                full((hid, hid)), full((1, hid)), full((hid, hid)), full((1, hid)),
                full((1, 1))],
      out_specs=[pl.BlockSpec((blk, hid), row), pl.BlockSpec((blk, 2 * hid), row),
                 pl.BlockSpec((blk, hid), row)],
      out_shape=out_shape,
  )(x_src, x_dst, W_emb, vec(b_emb), vec(lns_g), vec(lns_b), vec(lnd_g),
    vec(lnd_b), Wq, vec(bq), Wk, vec(bk), Wv, vec(bv), scale.reshape(1, 1))


def _eproj_body(ea, bd, be, e_out):
  e_out[...] = jnp.dot(ea[...], bd[...], preferred_element_type=jnp.float32) + be[...]


def _eproj(ea_packed, BD, be_tiled):
  rows, kdim = ea_packed.shape
  odim = BD.shape[1]
  blk = next(b for b in (1256, 1024, 1000, 628, 512, 256, 128, 64, 8)
             if rows % b == 0 and b % 8 == 0)
  return pl.pallas_call(
      _eproj_body,
      grid=(rows // blk,),
      in_specs=[pl.BlockSpec((blk, kdim), lambda i: (i, 0)),
                pl.BlockSpec((kdim, odim), lambda i: (0, 0)),
                pl.BlockSpec((1, odim), lambda i: (0, 0))],
      out_specs=pl.BlockSpec((blk, odim), lambda i: (i, 0)),
      out_shape=jax.ShapeDtypeStruct((rows, odim), jnp.float32),
  )(ea_packed, BD, be_tiled)


def _lane_perm(x, idx):
  return jax.lax.gather(
      x, idx[:, None],
      jax.lax.GatherDimensionNumbers(offset_dims=(), collapsed_slice_dims=(0,),
                                     start_index_map=(0,)),
      slice_sizes=(1,),
      mode=jax.lax.GatherScatterMode.PROMISE_IN_BOUNDS)


def _tree_sum(terms):
  while len(terms) > 1:
    nxt = [terms[i] + terms[i + 1] for i in range(0, len(terms) - 1, 2)]
    if len(terms) % 2:
      nxt.append(terms[-1])
    terms = nxt
  return terms[0]


def _make_edge_kernel(n_dst, n_edges_pad, hid, heads, chunk):
  dh = hid // heads
  ntiles = NC * NS
  nch = n_edges_pad // (ntiles * chunk)
  nrows = ((n_dst + 1 + NS - 1) // NS + 7) // 8 * 8
  acc_n = nrows * NS
  ndrows = (((n_dst + 1 + 15) // 16 + NS - 1) // NS + 7) // 8 * 8
  acc_nd = ndrows * NS
  groups = chunk // LANES

  mesh = plsc.VectorSubcoreMesh(
      core_axis_name="c", subcore_axis_name="s", num_cores=NC, num_subcores=NS)

  def body(sidx_hbm, didx_hbm, q_hbm, kv_hbm, e_hbm,
           pnum_hbm, pden_hbm,
           acc_num, acc_den,
           sidx0, sidx1, didx0, didx1, sdidx0, sdidx1, sd16_0, sd16_1,
           kv0, kv1, q0, q1, e0, e1, aden,
           sem_kv0, sem_kv1, sem_q0, sem_q1, sem_e0, sem_e1,
           sem_n0, sem_n1, sem_d):
    c = lax.axis_index("c")
    s = lax.axis_index("s")
    tile = c * NS + s
    r0 = s * nrows
    d0 = s * ndrows
    z16 = jnp.zeros((LANES,), jnp.float32)
    sidx = (sidx0, sidx1)
    didx = (didx0, didx1)
    sdidx = (sdidx0, sdidx1)
    sd16 = (sd16_0, sd16_1)
    kvb = (kv0, kv1)
    qb = (q0, q1)
    eb_ = (e0, e1)
    sem_kv = (sem_kv0, sem_kv1)
    sem_q = (sem_q0, sem_q1)
    sem_e = (sem_e0, sem_e1)
    sem_n = (sem_n0, sem_n1)

    def _zb(i, carry):
      for j in range(hid // LANES):
        q0[i, pl.ds(j * LANES, LANES)] = z16
        aden[i, pl.ds(j * LANES, LANES)] = z16
      return carry
    lax.fori_loop(0, chunk, _zb, 0)

    def _stripes(dst_at, base, total, src):
      nfull = total // chunk
      def _zs(i, carry):
        pltpu.sync_copy(src, dst_at(base + i * chunk, chunk))
        return carry
      lax.fori_loop(0, nfull, _zs, 0)
      rem = total - nfull * chunk
      off = base + nfull * chunk
      while rem >= 8:
        step = 16 if rem >= 16 else 8
        pltpu.sync_copy(src.at[pl.ds(0, step)], dst_at(off, step))
        off += step
        rem -= step

    _stripes(lambda r, n: acc_num.at[pl.ds(r, n)], r0, nrows, q0)
    _stripes(lambda r, n: acc_den.at[pl.ds(r, n)], d0, ndrows, aden)

    plsc.subcore_barrier()

    def _eb(i):
      return (tile + i * ntiles) * chunk

    def _idx_copy(i, b):
      pltpu.sync_copy(sidx_hbm.at[pl.ds(_eb(i), chunk)], sidx[b])
      pltpu.sync_copy(didx_hbm.at[pl.ds(_eb(i), chunk)], didx[b])

    def _issue_kvq(i, b):
      pltpu.async_copy(kv_hbm.at[sidx[b]], kvb[b], sem_kv[b])
      pltpu.async_copy(q_hbm.at[didx[b]], qb[b], sem_q[b])

    def _issue_e(i, b):
      pltpu.async_copy(e_hbm.at[pl.ds(_eb(i), chunk)], eb_[b], sem_e[b])

    _idx_copy(0, 0)
    _issue_kvq(0, 0)
    _issue_e(0, 0)
    _idx_copy(1, 1)
    _issue_kvq(1, 1)
    _issue_e(1, 1)

    def _outer(ci2, carry):
      for b in (0, 1):
        j = ci2 * 2 + b
        pltpu.make_async_copy(kv_hbm.at[sidx[b]], kvb[b], sem_kv[b]).wait()
        pltpu.make_async_copy(q_hbm.at[didx[b]], qb[b], sem_q[b]).wait()
        pltpu.make_async_copy(e_hbm.at[pl.ds(0, chunk)], eb_[b], sem_e[b]).wait()
        @pl.when(j >= 1)
        def _wd():
          pltpu.make_async_copy(aden, acc_den.at[sd16[1 - b]], sem_d).wait()
        def _za(i, carry2):
          for jj in range(hid // LANES):
            aden[i, pl.ds(jj * LANES, LANES)] = z16
          return carry2
        lax.fori_loop(0, chunk, _za, 0)
        def _sidx(g, carry2):
          iv = didx[b][pl.ds(g * LANES, LANES)]
          sdidx[b][pl.ds(g * LANES, LANES)] = iv
          sd16[b][pl.ds(g * LANES, LANES)] = lax.shift_right_logical(iv, 4)
          return carry2
        lax.fori_loop(0, groups, _sidx, 0)
        iota16 = lax.iota(jnp.int32, LANES)
        mask0 = iota16 == 0
        perm_idx = [iota16 ^ k for k in (8, 4, 2, 1)]
        def _edge(i, inner):
          blk = didx[b][pl.ds(i & ~(LANES - 1), LANES)]
          dsp = _lane_perm(blk, jnp.full((LANES,), i & (LANES - 1), jnp.int32))
          lanebase = (dsp & 15) * 8
          rowvec = jnp.full((LANES,), i, jnp.int32)
          for h in range(heads):
            sl = pl.ds(h * dh, LANES)
            qrow = qb[b][i, sl]
            krow = kvb[b][i, sl]
            erow = eb_[b][i, sl]
            vrow = kvb[b][i, pl.ds(hid + h * dh, LANES)]
            p = qrow * (krow + erow)
            for pidx in perm_idx:
              p = p + _lane_perm(p, pidx)
            ex = jnp.exp(p)
            plsc.store_scatter(aden, [rowvec, lanebase + h], ex, mask=mask0)
            eb_[b][i, sl] = (vrow + erow) * ex
          return inner
        lax.fori_loop(0, chunk, _edge, 0)
        pltpu.async_copy(eb_[b], acc_num.at[sdidx[b]], sem_n[b], add=True)
        pltpu.async_copy(aden, acc_den.at[sd16[b]], sem_d, add=True)
        @pl.when(j >= 1)
        def _wn():
          pltpu.make_async_copy(eb_[1 - b], acc_num.at[sdidx[1 - b]],
                                sem_n[1 - b]).wait()
        @pl.when(j + 2 < nch)
        def _st():
          _idx_copy(j + 2, b)
          _issue_kvq(j + 2, b)
        @pl.when((j >= 1) & (j + 1 < nch))
        def _se():
          _issue_e(j + 1, 1 - b)
      return carry

    lax.fori_loop(0, nch // 2, _outer, 0)

    lastb = (nch - 1) & 1
    pltpu.make_async_copy(eb_[lastb], acc_num.at[sdidx[lastb]], sem_n[lastb]).wait()
    pltpu.make_async_copy(aden, acc_den.at[sd16[lastb]], sem_d).wait()

    plsc.subcore_barrier()

    def _wout(acc, out_at, base, total, buf):
      nfull = total // chunk
      def _ws(i, carry):
        r = base + i * chunk
        pltpu.sync_copy(acc.at[pl.ds(r, chunk)], buf)
        pltpu.sync_copy(buf, out_at(r, chunk))
        return carry
      lax.fori_loop(0, nfull, _ws, 0)
      rem = total - nfull * chunk
      off = base + nfull * chunk
      while rem >= 8:
        step = 16 if rem >= 16 else 8
        pltpu.sync_copy(acc.at[pl.ds(off, step)], buf.at[pl.ds(0, step)])
        pltpu.sync_copy(buf.at[pl.ds(0, step)], out_at(off, step))
        off += step
        rem -= step

    _wout(acc_num, lambda r, n: pnum_hbm.at[c, pl.ds(r, n)], r0, nrows, q0)
    _wout(acc_den, lambda r, n: pden_hbm.at[c, pl.ds(r, n)], d0, ndrows, aden)

  f = pl.kernel(
      body,
      out_type=[jax.ShapeDtypeStruct((NC, acc_n, hid), jnp.float32),
                jax.ShapeDtypeStruct((NC, acc_nd, hid), jnp.float32)],
      mesh=mesh,
      compiler_params=pltpu.CompilerParams(needs_layout_passes=False),
      scratch_types=[
          pltpu.VMEM_SHARED((acc_n, hid), jnp.float32),
          pltpu.VMEM_SHARED((acc_nd, hid), jnp.float32),
          pltpu.VMEM((chunk,), jnp.int32), pltpu.VMEM((chunk,), jnp.int32),
          pltpu.VMEM((chunk,), jnp.int32), pltpu.VMEM((chunk,), jnp.int32),
          pltpu.VMEM((chunk,), jnp.int32), pltpu.VMEM((chunk,), jnp.int32),
          pltpu.VMEM((chunk,), jnp.int32), pltpu.VMEM((chunk,), jnp.int32),
          pltpu.VMEM((chunk, 2 * hid), jnp.float32),
          pltpu.VMEM((chunk, 2 * hid), jnp.float32),
          pltpu.VMEM((chunk, hid), jnp.float32),
          pltpu.VMEM((chunk, hid), jnp.float32),
          pltpu.VMEM((chunk, hid), jnp.float32),
          pltpu.VMEM((chunk, hid), jnp.float32),
          pltpu.VMEM((chunk, hid), jnp.float32),
          pltpu.SemaphoreType.DMA, pltpu.SemaphoreType.DMA,
          pltpu.SemaphoreType.DMA, pltpu.SemaphoreType.DMA,
          pltpu.SemaphoreType.DMA, pltpu.SemaphoreType.DMA,
          pltpu.SemaphoreType.DMA, pltpu.SemaphoreType.DMA,
          pltpu.SemaphoreType.DMA,
      ],
  )
  return f


def _post_body(pnum, pden, xde, r8, wo, bo, ln2g, ln2b, w1, b1, w2, b2, out):
  w = pnum[0] + pnum[1]
  den8 = pden[0] + pden[1]
  denw = jnp.dot(den8, r8[...], preferred_element_type=jnp.float32)
  agg = w / (denw + 1e-16)
  x1 = xde[...] + jnp.dot(agg, wo[...], preferred_element_type=jnp.float32) + bo[...]
  h = _ln(x1, ln2g[...], ln2b[...])
  mlp = jnp.dot(jax.nn.gelu(jnp.dot(h, w1[...], preferred_element_type=jnp.float32) + b1[...]),
                w2[...], preferred_element_type=jnp.float32) + b2[...]
  out[...] = x1 + mlp


def _post(pnum, pden, xde, R8, Wo, bo, ln2_g, ln2_b, W1, b1, W2, b2):
  n, hid = xde.shape
  mlp_hid = W1.shape[1]
  blk = 1000
  fix = lambda i: (0, 0)
  vec = lambda a: a.reshape(1, -1)
  return pl.pallas_call(
      _post_body,
      grid=(n // blk,),
      in_specs=[pl.BlockSpec((NC, blk, hid), lambda i: (0, i, 0)),
                pl.BlockSpec((NC, blk, 8), lambda i: (0, i, 0)),
                pl.BlockSpec((blk, hid), lambda i: (i, 0)),
                pl.BlockSpec((8, hid), fix),
                pl.BlockSpec((hid, hid), fix), pl.BlockSpec((1, hid), fix),
                pl.BlockSpec((1, hid), fix), pl.BlockSpec((1, hid), fix),
                pl.BlockSpec((hid, mlp_hid), fix), pl.BlockSpec((1, mlp_hid), fix),
                pl.BlockSpec((mlp_hid, hid), fix), pl.BlockSpec((1, hid), fix)],
      out_specs=pl.BlockSpec((blk, hid), lambda i: (i, 0)),
      out_shape=jax.ShapeDtypeStruct((n, hid), jnp.float32),
  )(pnum, pden, xde, R8, Wo, vec(bo), vec(ln2_g), vec(ln2_b), W1, vec(b1), W2, vec(b2))


def kernel(x_src, x_dst, edge_attr, edge_index, batch_size,
           W_emb, b_emb, lns_g, lns_b, lnd_g, lnd_b,
           Wq, bq, Wk, bk, Wv, bv, We, be, Wo, bo,
           ln2_g, ln2_b, W1, b1, W2, b2):
  n_src, hid = x_src.shape
  n_dst = x_dst.shape[0]
  n_edges, edge_dim = edge_attr.shape
  heads = 8
  dh = hid // heads
  pack = hid // edge_dim
  chunk = 32
  ntiles = NC * NS

  nch = -(-n_edges // (ntiles * chunk))
  if nch % 2:
    nch += 1
  n_edges_pad = nch * ntiles * chunk

  scale = jnp.float32(1.0) / jnp.sqrt(jnp.float32(dh))
  q, kv, xde = _prep(x_src, x_dst, W_emb, b_emb, lns_g, lns_b, lnd_g, lnd_b,
                     Wq, bq, Wk, bk, Wv, bv, scale)
  q = jnp.pad(q, ((0, 8), (0, 0)))
  kv = jnp.pad(kv, ((0, 8), (0, 0)))

  BD = jnp.einsum("ij,ao->iajo", jnp.eye(pack, dtype=jnp.float32),
                  We).reshape(pack * edge_dim, pack * hid)
  be_tiled = jnp.tile(be, pack).reshape(1, pack * hid)
  ea_packed = edge_attr.reshape(n_edges // pack, pack * edge_dim)
  ea_packed = jnp.pad(ea_packed, ((0, (n_edges_pad - n_edges) // pack), (0, 0)))
  e = _eproj(ea_packed, BD, be_tiled).reshape(n_edges_pad, hid)

  ei = edge_index.astype(jnp.int32)
  src_idx = jnp.pad(ei[0], (0, n_edges_pad - n_edges))
  dst_idx = jnp.pad(ei[1], (0, n_edges_pad - n_edges),
                    constant_values=n_dst)

  edge_f = _make_edge_kernel(n_dst, n_edges_pad, hid, heads, chunk=chunk)
  pnum, pden = edge_f(src_idx, dst_idx, q, kv, e)
  den8 = pden.reshape(pden.shape[0], pden.shape[1] * (hid // 8), 8)

  cols = jnp.arange(hid) // dh
  R8 = (jnp.arange(8)[:, None] == cols[None, :]).astype(jnp.float32)

  return _post(pnum, den8, xde, R8, Wo, bo, ln2_g, ln2_b, W1, b1, W2, b2)


# speedup vs baseline: 32.1457x; 2.1291x over previous
import functools

import jax
import jax.numpy as jnp
from jax import lax
from jax.experimental import pallas as pl
from jax.experimental.pallas import tpu as pltpu
from jax.experimental.pallas import tpu_sc as plsc

NC = 2
NS = 16
LANES = 16
ACC_W = 144


def _ln(x, g, b):
  m = jnp.mean(x, axis=-1, keepdims=True)
  v = jnp.mean((x - m) ** 2, axis=-1, keepdims=True)
  return (x - m) / jnp.sqrt(v + 1e-5) * g + b


def _prep_body(xsrc, xdst, wemb, bemb, lnsg, lnsb, lndg, lndb,
               wq, bq, wk, bk, wv, bv, scale_ref,
               q_out, kv_out, xde_out):
  xde = jnp.dot(xdst[...], wemb[...], preferred_element_type=jnp.float32) + bemb[...]
  xde_out[...] = xde
  xs = _ln(xsrc[...], lnsg[...], lnsb[...])
  xd = _ln(xde, lndg[...], lndb[...])
  sc = scale_ref[0, 0]
  hid = xsrc.shape[1]
  q_out[...] = (jnp.dot(xd, wq[...], preferred_element_type=jnp.float32) + bq[...]) * sc
  kv_out[:, 0:hid] = jnp.dot(xs, wk[...], preferred_element_type=jnp.float32) + bk[...]
  kv_out[:, hid:2 * hid] = jnp.dot(xs, wv[...], preferred_element_type=jnp.float32) + bv[...]


def _prep(x_src, x_dst, W_emb, b_emb, lns_g, lns_b, lnd_g, lnd_b,
          Wq, bq, Wk, bk, Wv, bv, scale):
  n, hid = x_src.shape
  blk = 1000
  grid = (n // blk,)
  row = lambda i: (i, 0)
  fix = lambda i: (0, 0)
  vec = lambda a: a.reshape(1, -1)
  full = lambda shp: pl.BlockSpec(shp, fix)
  out_shape = [jax.ShapeDtypeStruct((n, hid), jnp.float32),
               jax.ShapeDtypeStruct((n, 2 * hid), jnp.float32),
               jax.ShapeDtypeStruct((n, hid), jnp.float32)]
  return pl.pallas_call(
      _prep_body,
      grid=grid,
      in_specs=[pl.BlockSpec((blk, hid), row), pl.BlockSpec((blk, hid), row),
                full((hid, hid)), full((1, hid)), full((1, hid)), full((1, hid)),
                full((1, hid)), full((1, hid)), full((hid, hid)), full((1, hid)),
                full((hid, hid)), full((1, hid)), full((hid, hid)), full((1, hid)),
                full((1, 1))],
      out_specs=[pl.BlockSpec((blk, hid), row), pl.BlockSpec((blk, 2 * hid), row),
                 pl.BlockSpec((blk, hid), row)],
      out_shape=out_shape,
  )(x_src, x_dst, W_emb, vec(b_emb), vec(lns_g), vec(lns_b), vec(lnd_g),
    vec(lnd_b), Wq, vec(bq), Wk, vec(bk), Wv, vec(bv), scale.reshape(1, 1))


def _eproj_body(ea, bd, be, e_out):
  e_out[...] = jnp.dot(ea[...], bd[...], preferred_element_type=jnp.float32) + be[...]


def _eproj(ea_packed, BD, be_tiled):
  rows, kdim = ea_packed.shape
  odim = BD.shape[1]
  blk = next(b for b in (1256, 1024, 1000, 628, 512, 256, 128, 64, 8)
             if rows % b == 0 and b % 8 == 0)
  return pl.pallas_call(
      _eproj_body,
      grid=(rows // blk,),
      in_specs=[pl.BlockSpec((blk, kdim), lambda i: (i, 0)),
                pl.BlockSpec((kdim, odim), lambda i: (0, 0)),
                pl.BlockSpec((1, odim), lambda i: (0, 0))],
      out_specs=pl.BlockSpec((blk, odim), lambda i: (i, 0)),
      out_shape=jax.ShapeDtypeStruct((rows, odim), jnp.float32),
  )(ea_packed, BD, be_tiled)


def _lane_perm(x, idx):
  return jax.lax.gather(
      x, idx[:, None],
      jax.lax.GatherDimensionNumbers(offset_dims=(), collapsed_slice_dims=(0,),
                                     start_index_map=(0,)),
      slice_sizes=(1,),
      mode=jax.lax.GatherScatterMode.PROMISE_IN_BOUNDS)


def _tree_sum(terms):
  while len(terms) > 1:
    nxt = [terms[i] + terms[i + 1] for i in range(0, len(terms) - 1, 2)]
    if len(terms) % 2:
      nxt.append(terms[-1])
    terms = nxt
  return terms[0]


def _make_edge_kernel(n_dst, n_edges_pad, hid, heads, chunk):
  dh = hid // heads
  ntiles = NC * NS
  nch = n_edges_pad // (ntiles * chunk)
  nrows = ((n_dst + 1 + NS - 1) // NS + 7) // 8 * 8
  acc_n = nrows * NS
  ndrows = (((n_dst + 1 + 15) // 16 + NS - 1) // NS + 7) // 8 * 8
  acc_nd = ndrows * NS
  groups = chunk // LANES

  mesh = plsc.VectorSubcoreMesh(
      core_axis_name="c", subcore_axis_name="s", num_cores=NC, num_subcores=NS)

  def body(sidx_hbm, didx_hbm, q_hbm, kv_hbm, e_hbm,
           pnum_hbm, pden_hbm,
           acc_num, acc_den,
           sidx0, sidx1, sidx2, sidx3, didx0, didx1, didx2, didx3,
           sdidx0, sdidx1, sd16_0, sd16_1,
           kv0, kv1, q0, q1, e0, e1, aden,
           sem_i0, sem_i1, sem_i2, sem_i3,
           sem_kv0, sem_kv1, sem_q0, sem_q1, sem_e0, sem_e1,
           sem_n0, sem_n1, sem_d):
    c = lax.axis_index("c")
    s = lax.axis_index("s")
    tile = c * NS + s
    r0 = s * nrows
    d0 = s * ndrows
    z16 = jnp.zeros((LANES,), jnp.float32)
    sidx = (sidx0, sidx1, sidx2, sidx3)
    didx = (didx0, didx1, didx2, didx3)
    sem_i = (sem_i0, sem_i1, sem_i2, sem_i3)
    sdidx = (sdidx0, sdidx1)
    sd16 = (sd16_0, sd16_1)
    kvb = (kv0, kv1)
    qb = (q0, q1)
    eb_ = (e0, e1)
    sem_kv = (sem_kv0, sem_kv1)
    sem_q = (sem_q0, sem_q1)
    sem_e = (sem_e0, sem_e1)
    sem_n = (sem_n0, sem_n1)

    def _zb(i, carry):
      for j in range(hid // LANES):
        q0[i, pl.ds(j * LANES, LANES)] = z16
        aden[i, pl.ds(j * LANES, LANES)] = z16
      return carry
    lax.fori_loop(0, chunk, _zb, 0)

    def _stripes(dst_at, base, total, src):
      nfull = total // chunk
      def _zs(i, carry):
        pltpu.sync_copy(src, dst_at(base + i * chunk, chunk))
        return carry
      lax.fori_loop(0, nfull, _zs, 0)
      rem = total - nfull * chunk
      off = base + nfull * chunk
      while rem >= 8:
        step = 16 if rem >= 16 else 8
        pltpu.sync_copy(src.at[pl.ds(0, step)], dst_at(off, step))
        off += step
        rem -= step

    _stripes(lambda r, n: acc_num.at[pl.ds(r, n)], r0, nrows, q0)
    _stripes(lambda r, n: acc_den.at[pl.ds(r, n)], d0, ndrows, aden)

    plsc.subcore_barrier()

    def _eb(i):
      return (tile + i * ntiles) * chunk

    def _idx_copy_sync(i, s4):
      pltpu.sync_copy(sidx_hbm.at[pl.ds(_eb(i), chunk)], sidx[s4])
      pltpu.sync_copy(didx_hbm.at[pl.ds(_eb(i), chunk)], didx[s4])

    def _idx_copy_async(i, s4):
      pltpu.async_copy(sidx_hbm.at[pl.ds(_eb(i), chunk)], sidx[s4], sem_i[s4])
      pltpu.async_copy(didx_hbm.at[pl.ds(_eb(i), chunk)], didx[s4], sem_i[s4])

    def _wait_idx(s4):
      pltpu.make_async_copy(sidx_hbm.at[pl.ds(0, chunk)], sidx[s4], sem_i[s4]).wait()
      pltpu.make_async_copy(didx_hbm.at[pl.ds(0, chunk)], didx[s4], sem_i[s4]).wait()

    def _issue_kvq(s4, p):
      pltpu.async_copy(kv_hbm.at[sidx[s4]], kvb[p], sem_kv[p])
      pltpu.async_copy(q_hbm.at[didx[s4]], qb[p], sem_q[p])

    def _issue_e(i, p):
      pltpu.async_copy(e_hbm.at[pl.ds(_eb(i), chunk)], eb_[p], sem_e[p])

    def _copy_sdidx(s4, p):
      def _sc(g, carry2):
        iv = didx[s4][pl.ds(g * LANES, LANES)]
        sdidx[p][pl.ds(g * LANES, LANES)] = iv
        sd16[p][pl.ds(g * LANES, LANES)] = lax.shift_right_logical(iv, 4)
        return carry2
      lax.fori_loop(0, groups, _sc, 0)

    _idx_copy_sync(0, 0)
    _idx_copy_sync(1, 1)
    _idx_copy_async(2, 2)
    _issue_kvq(0, 0)
    _issue_e(0, 0)
    _issue_kvq(1, 1)
    _copy_sdidx(0, 0)

    iota16 = lax.iota(jnp.int32, LANES)
    mask0 = iota16 == 0
    perm_idx = [iota16 ^ k for k in (8, 4, 2, 1)]

    def _outer(ci4, carry):
      for b4 in (0, 1, 2, 3):
        j = ci4 * 4 + b4
        p = b4 & 1
        s4 = b4 & 3
        pltpu.make_async_copy(kv_hbm.at[sidx[s4]], kvb[p], sem_kv[p]).wait()
        pltpu.make_async_copy(q_hbm.at[didx[s4]], qb[p], sem_q[p]).wait()
        pltpu.make_async_copy(e_hbm.at[pl.ds(0, chunk)], eb_[p], sem_e[p]).wait()
        @pl.when(j >= 1)
        def _wd():
          pltpu.make_async_copy(aden, acc_den.at[sd16[1 - p]], sem_d).wait()
        def _za(i, carry2):
          for jj in range(hid // LANES):
            aden[i, pl.ds(jj * LANES, LANES)] = z16
          return carry2
        lax.fori_loop(0, chunk, _za, 0)
        def _edge(i, inner):
          blk = didx[s4][pl.ds(i & ~(LANES - 1), LANES)]
          dsp = _lane_perm(blk, jnp.full((LANES,), i & (LANES - 1), jnp.int32))
          lanebase = (dsp & 15) * 8
          rowvec = jnp.full((LANES,), i, jnp.int32)
          qs = [qb[p][i, pl.ds(h * dh, LANES)] for h in range(heads)]
          ks = [kvb[p][i, pl.ds(h * dh, LANES)] for h in range(heads)]
          es = [eb_[p][i, pl.ds(h * dh, LANES)] for h in range(heads)]
          vs = [kvb[p][i, pl.ds(hid + h * dh, LANES)] for h in range(heads)]
          ps = [qs[h] * (ks[h] + es[h]) for h in range(heads)]
          for pidx in perm_idx:
            ps = [pp + _lane_perm(pp, pidx) for pp in ps]
          exs = [jnp.exp(pp) for pp in ps]
          for h in range(heads):
            plsc.store_scatter(aden, [rowvec, lanebase + h], exs[h], mask=mask0)
          for h in range(heads):
            eb_[p][i, pl.ds(h * dh, LANES)] = (vs[h] + es[h]) * exs[h]
          return inner
        lax.fori_loop(0, chunk, _edge, 0)
        pltpu.async_copy(eb_[p], acc_num.at[sdidx[p]], sem_n[p], add=True)
        pltpu.async_copy(aden, acc_den.at[sd16[p]], sem_d, add=True)
        @pl.when(j >= 1)
        def _wn():
          pltpu.make_async_copy(eb_[1 - p], acc_num.at[sdidx[1 - p]],
                                sem_n[1 - p]).wait()
        @pl.when(j + 1 < nch)
        def _se():
          _issue_e(j + 1, 1 - p)
          _copy_sdidx((s4 + 1) & 3, 1 - p)
        @pl.when(j + 2 < nch)
        def _st():
          _wait_idx((s4 + 2) & 3)
          _issue_kvq((s4 + 2) & 3, p)
        @pl.when(j + 3 < nch)
        def _si():
          _idx_copy_async(j + 3, (s4 + 3) & 3)
      return carry

    lax.fori_loop(0, nch // 4, _outer, 0)

    lastp = (nch - 1) & 1
    pltpu.make_async_copy(eb_[lastp], acc_num.at[sdidx[lastp]], sem_n[lastp]).wait()
    pltpu.make_async_copy(aden, acc_den.at[sd16[lastp]], sem_d).wait()

    plsc.subcore_barrier()

    def _wout(acc, out_at, base, total, buf):
      nfull = total // chunk
      def _ws(i, carry):
        r = base + i * chunk
        pltpu.sync_copy(acc.at[pl.ds(r, chunk)], buf)
        pltpu.sync_copy(buf, out_at(r, chunk))
        return carry
      lax.fori_loop(0, nfull, _ws, 0)
      rem = total - nfull * chunk
      off = base + nfull * chunk
      while rem >= 8:
        step = 16 if rem >= 16 else 8
        pltpu.sync_copy(acc.at[pl.ds(off, step)], buf.at[pl.ds(0, step)])
        pltpu.sync_copy(buf.at[pl.ds(0, step)], out_at(off, step))
        off += step
        rem -= step

    _wout(acc_num, lambda r, n: pnum_hbm.at[c, pl.ds(r, n)], r0, nrows, q0)
    _wout(acc_den, lambda r, n: pden_hbm.at[c, pl.ds(r, n)], d0, ndrows, aden)

  f = pl.kernel(
      body,
      out_type=[jax.ShapeDtypeStruct((NC, acc_n, hid), jnp.float32),
                jax.ShapeDtypeStruct((NC, acc_nd, hid), jnp.float32)],
      mesh=mesh,
      compiler_params=pltpu.CompilerParams(needs_layout_passes=False),
      scratch_types=[
          pltpu.VMEM_SHARED((acc_n, hid), jnp.float32),
          pltpu.VMEM_SHARED((acc_nd, hid), jnp.float32),
          pltpu.VMEM((chunk,), jnp.int32), pltpu.VMEM((chunk,), jnp.int32),
          pltpu.VMEM((chunk,), jnp.int32), pltpu.VMEM((chunk,), jnp.int32),
          pltpu.VMEM((chunk,), jnp.int32), pltpu.VMEM((chunk,), jnp.int32),
          pltpu.VMEM((chunk,), jnp.int32), pltpu.VMEM((chunk,), jnp.int32),
          pltpu.VMEM((chunk,), jnp.int32), pltpu.VMEM((chunk,), jnp.int32),
          pltpu.VMEM((chunk,), jnp.int32), pltpu.VMEM((chunk,), jnp.int32),
          pltpu.VMEM((chunk, 2 * hid), jnp.float32),
          pltpu.VMEM((chunk, 2 * hid), jnp.float32),
          pltpu.VMEM((chunk, hid), jnp.float32),
          pltpu.VMEM((chunk, hid), jnp.float32),
          pltpu.VMEM((chunk, hid), jnp.float32),
          pltpu.VMEM((chunk, hid), jnp.float32),
          pltpu.VMEM((chunk, hid), jnp.float32),
          pltpu.SemaphoreType.DMA, pltpu.SemaphoreType.DMA,
          pltpu.SemaphoreType.DMA, pltpu.SemaphoreType.DMA,
          pltpu.SemaphoreType.DMA, pltpu.SemaphoreType.DMA,
          pltpu.SemaphoreType.DMA, pltpu.SemaphoreType.DMA,
          pltpu.SemaphoreType.DMA, pltpu.SemaphoreType.DMA,
          pltpu.SemaphoreType.DMA, pltpu.SemaphoreType.DMA,
          pltpu.SemaphoreType.DMA,
      ],
  )
  return f


def _post_body(pnum, pden, xde, r8, wo, bo, ln2g, ln2b, w1, b1, w2, b2, out):
  w = pnum[0] + pnum[1]
  den8 = pden[0] + pden[1]
  denw = jnp.dot(den8, r8[...], preferred_element_type=jnp.float32)
  agg = w / (denw + 1e-16)
  x1 = xde[...] + jnp.dot(agg, wo[...], preferred_element_type=jnp.float32) + bo[...]
  h = _ln(x1, ln2g[...], ln2b[...])
  mlp = jnp.dot(jax.nn.gelu(jnp.dot(h, w1[...], preferred_element_type=jnp.float32) + b1[...]),
                w2[...], preferred_element_type=jnp.float32) + b2[...]
  out[...] = x1 + mlp


def _post(pnum, pden, xde, R8, Wo, bo, ln2_g, ln2_b, W1, b1, W2, b2):
  n, hid = xde.shape
  mlp_hid = W1.shape[1]
  blk = 1000
  fix = lambda i: (0, 0)
  vec = lambda a: a.reshape(1, -1)
  return pl.pallas_call(
      _post_body,
      grid=(n // blk,),
      in_specs=[pl.BlockSpec((NC, blk, hid), lambda i: (0, i, 0)),
                pl.BlockSpec((NC, blk, 8), lambda i: (0, i, 0)),
                pl.BlockSpec((blk, hid), lambda i: (i, 0)),
                pl.BlockSpec((8, hid), fix),
                pl.BlockSpec((hid, hid), fix), pl.BlockSpec((1, hid), fix),
                pl.BlockSpec((1, hid), fix), pl.BlockSpec((1, hid), fix),
                pl.BlockSpec((hid, mlp_hid), fix), pl.BlockSpec((1, mlp_hid), fix),
                pl.BlockSpec((mlp_hid, hid), fix), pl.BlockSpec((1, hid), fix)],
      out_specs=pl.BlockSpec((blk, hid), lambda i: (i, 0)),
      out_shape=jax.ShapeDtypeStruct((n, hid), jnp.float32),
  )(pnum, pden, xde, R8, Wo, vec(bo), vec(ln2_g), vec(ln2_b), W1, vec(b1), W2, vec(b2))


def kernel(x_src, x_dst, edge_attr, edge_index, batch_size,
           W_emb, b_emb, lns_g, lns_b, lnd_g, lnd_b,
           Wq, bq, Wk, bk, Wv, bv, We, be, Wo, bo,
           ln2_g, ln2_b, W1, b1, W2, b2):
  n_src, hid = x_src.shape
  n_dst = x_dst.shape[0]
  n_edges, edge_dim = edge_attr.shape
  heads = 8
  dh = hid // heads
  pack = hid // edge_dim
  chunk = 32
  ntiles = NC * NS

  nch = -(-n_edges // (ntiles * chunk))
  nch = (nch + 3) // 4 * 4
  n_edges_pad = nch * ntiles * chunk

  scale = jnp.float32(1.0) / jnp.sqrt(jnp.float32(dh))
  q, kv, xde = _prep(x_src, x_dst, W_emb, b_emb, lns_g, lns_b, lnd_g, lnd_b,
                     Wq, bq, Wk, bk, Wv, bv, scale)
  q = jnp.pad(q, ((0, 8), (0, 0)))
  kv = jnp.pad(kv, ((0, 8), (0, 0)))

  BD = jnp.einsum("ij,ao->iajo", jnp.eye(pack, dtype=jnp.float32),
                  We).reshape(pack * edge_dim, pack * hid)
  be_tiled = jnp.tile(be, pack).reshape(1, pack * hid)
  ea_packed = edge_attr.reshape(n_edges // pack, pack * edge_dim)
  ea_packed = jnp.pad(ea_packed, ((0, (n_edges_pad - n_edges) // pack), (0, 0)))
  e = _eproj(ea_packed, BD, be_tiled).reshape(n_edges_pad, hid)

  ei = edge_index.astype(jnp.int32)
  src_idx = jnp.pad(ei[0], (0, n_edges_pad - n_edges))
  dst_idx = jnp.pad(ei[1], (0, n_edges_pad - n_edges),
                    constant_values=n_dst)

  edge_f = _make_edge_kernel(n_dst, n_edges_pad, hid, heads, chunk=chunk)
  pnum, pden = edge_f(src_idx, dst_idx, q, kv, e)
  den8 = pden.reshape(pden.shape[0], pden.shape[1] * (hid // 8), 8)

  cols = jnp.arange(hid) // dh
  R8 = (jnp.arange(8)[:, None] == cols[None, :]).astype(jnp.float32)

  return _post(pnum, den8, xde, R8, Wo, bo, ln2_g, ln2_b, W1, b1, W2, b2)
